# Initial kernel scaffold; baseline (speedup 1.0000x reference)
import jax
import jax.numpy as jnp
from jax.experimental import pallas as pl


def kernel(x, graph_index, W_enc, b_enc, W_head, b_head):
    raise NotImplementedError("write your pallas kernel here")


# speedup vs baseline: 8.6077x; 8.6077x over previous
import functools

import jax
import jax.numpy as jnp
from jax import lax
from jax.experimental import pallas as pl
from jax.experimental.pallas import tpu as pltpu
from jax.experimental.pallas import tpu_sc as plsc

N = 6_400_000
D = 4
DW = 8
G = 100_000

NC = 2
NS = 16
NW = NC * NS

GROUP = 125
K = 16
PASS_ROWS = K * GROUP
GROUPS_TOTAL = N // GROUP
GROUPS_PER_TILE = GROUPS_TOTAL // NW
PASSES = GROUPS_PER_TILE // K
ROWS_PER_TILE = N // NW

GP = 100_352
CHUNK = GP // NS
SUB = CHUNK // 4


def _sc_body(xw_hbm, idxg_hbm, z8_hbm, out_s_hbm,
             xbuf, ibuf, st8, acc_s):
    cid = lax.axis_index("c")
    sid = lax.axis_index("s")
    wid = sid * NC + cid

    woff = sid * CHUNK
    pltpu.sync_copy(z8_hbm, st8)
    for q in range(4):
        off = woff + q * SUB
        pltpu.sync_copy(st8, acc_s.at[pl.ds(off, SUB), :])
    plsc.subcore_barrier()

    def one_pass(p, carry):
        r0 = wid * ROWS_PER_TILE + p * PASS_ROWS
        g0 = wid * GROUPS_PER_TILE + p * K
        pltpu.sync_copy(xw_hbm.at[pl.ds(r0, PASS_ROWS), :], xbuf)
        pltpu.sync_copy(idxg_hbm.at[pl.ds(g0, K), :], ibuf)

        def one_group(j, c2):
            pltpu.sync_copy(xbuf.at[pl.ds(j * GROUP, GROUP), :],
                            acc_s.at[ibuf.at[j]], add=True)
            return c2

        lax.fori_loop(0, K, one_group, 0)
        return carry

    lax.fori_loop(0, PASSES, one_pass, 0)
    plsc.subcore_barrier()

    for q in range(4):
        off = woff + q * SUB
        pltpu.sync_copy(acc_s.at[pl.ds(off, SUB), :], st8)
        pltpu.sync_copy(st8, out_s_hbm.at[cid, pl.ds(off, SUB), :])


@functools.cache
def _make_sc_segsum():
    return pl.kernel(
        _sc_body,
        out_type=jax.ShapeDtypeStruct((NC, GP, DW), jnp.float32),
        mesh=plsc.VectorSubcoreMesh(core_axis_name="c", subcore_axis_name="s",
                                    num_cores=NC, num_subcores=NS),
        scratch_types=[
            pltpu.VMEM((PASS_ROWS, DW), jnp.float32),
            pltpu.VMEM((K, GROUP), jnp.int32),
            pltpu.VMEM((SUB, DW), jnp.float32),
            pltpu.VMEM_SHARED((GP, DW), jnp.float32),
        ],
        compiler_params=pltpu.CompilerParams(use_tc_tiling_on_sc=False),
    )


BG = 6272


def _finalize_body(ps_ref, wet_ref, e4_ref, be_ref, wht_ref, bh_ref, out_ref):
    s8 = ps_ref[0] + ps_ref[1]
    c = jnp.dot(s8, e4_ref[...],
                preferred_element_type=jnp.float32)
    denom = jnp.maximum(c, 1.0)
    gr = (jnp.dot(s8, wet_ref[...], preferred_element_type=jnp.float32)
          + c * be_ref[...]) / denom
    out_ref[...] = (jnp.dot(gr, wht_ref[...], preferred_element_type=jnp.float32)
                    + bh_ref[...])


def _finalize(psums, wet8, e4, be, wht, bh):
    return pl.pallas_call(
        _finalize_body,
        grid=(GP // BG,),
        in_specs=[
            pl.BlockSpec((NC, BG, DW), lambda i: (0, i, 0)),
            pl.BlockSpec((DW, D), lambda i: (0, 0)),
            pl.BlockSpec((DW, 1), lambda i: (0, 0)),
            pl.BlockSpec((1, D), lambda i: (0, 0)),
            pl.BlockSpec((D, 2), lambda i: (0, 0)),
            pl.BlockSpec((1, 2), lambda i: (0, 0)),
        ],
        out_specs=pl.BlockSpec((BG, 2), lambda i: (i, 0)),
        out_shape=jax.ShapeDtypeStruct((GP, 2), jnp.float32),
    )(psums, wet8, e4, be, wht, bh)


def kernel(x, graph_index, W_enc, b_enc, W_head, b_head):
    xw = jnp.concatenate(
        [x, jnp.ones((N, 1), jnp.float32), jnp.zeros((N, 3), jnp.float32)],
        axis=1)
    idxg = graph_index.reshape(GROUPS_TOTAL, GROUP)
    z8 = jnp.zeros((SUB, DW), jnp.float32)
    psums = _make_sc_segsum()(xw, idxg, z8)
    wet8 = jnp.zeros((DW, D), jnp.float32).at[:D].set(W_enc.T)
    e4 = jnp.zeros((DW, 1), jnp.float32).at[D].set(1.0)
    out_full = _finalize(psums, wet8, e4,
                         b_enc.reshape(1, D),
                         W_head.T, b_head.reshape(1, 2))
    return out_full[:G]


# speedup vs baseline: 9.0909x; 1.0561x over previous
import functools

import jax
import jax.numpy as jnp
from jax import lax
from jax.experimental import pallas as pl
from jax.experimental.pallas import tpu as pltpu
from jax.experimental.pallas import tpu_sc as plsc

N = 6_400_000
D = 4
DW = 8
G = 100_000

NC = 2
NS = 16
NW = NC * NS

GROUP = 125
K = 16
PASS_ROWS = K * GROUP
GROUPS_TOTAL = N // GROUP
GROUPS_PER_TILE = GROUPS_TOTAL // NW
PASSES = GROUPS_PER_TILE // K
ROWS_PER_TILE = N // NW

GP = 100_352
CHUNK = GP // NS
SUB = CHUNK // 4


def _sc_body(xw_hbm, idxg_hbm, z8_hbm, out_s_hbm,
             xbuf0, ibuf0, xbuf1, ibuf1, st8, acc_s,
             ssem, xsem0, isem0, xsem1, isem1):
    cid = lax.axis_index("c")
    sid = lax.axis_index("s")
    wid = sid * NC + cid

    woff = sid * CHUNK
    pltpu.sync_copy(z8_hbm, st8)
    for q in range(4):
        off = woff + q * SUB
        pltpu.sync_copy(st8, acc_s.at[pl.ds(off, SUB), :])
    plsc.subcore_barrier()

    def stage(p, xbuf, ibuf, xsem, isem):
        r0 = wid * ROWS_PER_TILE + p * PASS_ROWS
        g0 = wid * GROUPS_PER_TILE + p * K
        pltpu.async_copy(xw_hbm.at[pl.ds(r0, PASS_ROWS), :], xbuf, xsem)
        pltpu.async_copy(idxg_hbm.at[pl.ds(g0, K), :], ibuf, isem)

    def stage_wait(p, xbuf, ibuf, xsem, isem):
        r0 = wid * ROWS_PER_TILE + p * PASS_ROWS
        g0 = wid * GROUPS_PER_TILE + p * K
        pltpu.make_async_copy(xw_hbm.at[pl.ds(r0, PASS_ROWS), :],
                              xbuf, xsem).wait()
        pltpu.make_async_copy(idxg_hbm.at[pl.ds(g0, K), :],
                              ibuf, isem).wait()

    def scatter(xbuf, ibuf):
        def fire_group(j, c2):
            pltpu.async_copy(xbuf.at[pl.ds(j * GROUP, GROUP), :],
                             acc_s.at[ibuf.at[j]], ssem, add=True)
            return c2

        def drain_group(j, c2):
            pltpu.make_async_copy(xbuf.at[pl.ds(j * GROUP, GROUP), :],
                                  acc_s.at[ibuf.at[j]], ssem).wait()
            return c2

        lax.fori_loop(0, K, fire_group, 0)
        lax.fori_loop(0, K, drain_group, 0)

    stage(0, xbuf0, ibuf0, xsem0, isem0)

    def two_passes(i, carry):
        p0 = 2 * i
        p1 = p0 + 1
        p2 = jnp.minimum(p0 + 2, PASSES - 1)
        stage_wait(p0, xbuf0, ibuf0, xsem0, isem0)
        stage(p1, xbuf1, ibuf1, xsem1, isem1)
        scatter(xbuf0, ibuf0)
        stage_wait(p1, xbuf1, ibuf1, xsem1, isem1)
        stage(p2, xbuf0, ibuf0, xsem0, isem0)
        scatter(xbuf1, ibuf1)
        return carry

    lax.fori_loop(0, PASSES // 2, two_passes, 0)
    stage_wait(PASSES - 1, xbuf0, ibuf0, xsem0, isem0)
    plsc.subcore_barrier()

    for q in range(4):
        off = woff + q * SUB
        pltpu.sync_copy(acc_s.at[pl.ds(off, SUB), :], st8)
        pltpu.sync_copy(st8, out_s_hbm.at[cid, pl.ds(off, SUB), :])


@functools.cache
def _make_sc_segsum():
    return pl.kernel(
        _sc_body,
        out_type=jax.ShapeDtypeStruct((NC, GP, DW), jnp.float32),
        mesh=plsc.VectorSubcoreMesh(core_axis_name="c", subcore_axis_name="s",
                                    num_cores=NC, num_subcores=NS),
        scratch_types=[
            pltpu.VMEM((PASS_ROWS, DW), jnp.float32),
            pltpu.VMEM((K, GROUP), jnp.int32),
            pltpu.VMEM((PASS_ROWS, DW), jnp.float32),
            pltpu.VMEM((K, GROUP), jnp.int32),
            pltpu.VMEM((SUB, DW), jnp.float32),
            pltpu.VMEM_SHARED((GP, DW), jnp.float32),
            pltpu.SemaphoreType.DMA,
            pltpu.SemaphoreType.DMA,
            pltpu.SemaphoreType.DMA,
            pltpu.SemaphoreType.DMA,
            pltpu.SemaphoreType.DMA,
        ],
        compiler_params=pltpu.CompilerParams(use_tc_tiling_on_sc=False),
    )


BG = 6272


def _finalize_body(ps_ref, wet_ref, e4_ref, be_ref, wht_ref, bh_ref, out_ref):
    s8 = ps_ref[0] + ps_ref[1]
    c = jnp.dot(s8, e4_ref[...],
                preferred_element_type=jnp.float32)
    denom = jnp.maximum(c, 1.0)
    gr = (jnp.dot(s8, wet_ref[...], preferred_element_type=jnp.float32)
          + c * be_ref[...]) / denom
    out_ref[...] = (jnp.dot(gr, wht_ref[...], preferred_element_type=jnp.float32)
                    + bh_ref[...])


def _finalize(psums, wet8, e4, be, wht, bh):
    return pl.pallas_call(
        _finalize_body,
        grid=(GP // BG,),
        in_specs=[
            pl.BlockSpec((NC, BG, DW), lambda i: (0, i, 0)),
            pl.BlockSpec((DW, D), lambda i: (0, 0)),
            pl.BlockSpec((DW, 1), lambda i: (0, 0)),
            pl.BlockSpec((1, D), lambda i: (0, 0)),
            pl.BlockSpec((D, 2), lambda i: (0, 0)),
            pl.BlockSpec((1, 2), lambda i: (0, 0)),
        ],
        out_specs=pl.BlockSpec((BG, 2), lambda i: (i, 0)),
        out_shape=jax.ShapeDtypeStruct((GP, 2), jnp.float32),
    )(psums, wet8, e4, be, wht, bh)


def kernel(x, graph_index, W_enc, b_enc, W_head, b_head):
    xw = jnp.concatenate(
        [x, jnp.ones((N, 1), jnp.float32), jnp.zeros((N, 3), jnp.float32)],
        axis=1)
    idxg = graph_index.reshape(GROUPS_TOTAL, GROUP)
    z8 = jnp.zeros((SUB, DW), jnp.float32)
    psums = _make_sc_segsum()(xw, idxg, z8)
    wet8 = jnp.zeros((DW, D), jnp.float32).at[:D].set(W_enc.T)
    e4 = jnp.zeros((DW, 1), jnp.float32).at[D].set(1.0)
    out_full = _finalize(psums, wet8, e4,
                         b_enc.reshape(1, D),
                         W_head.T, b_head.reshape(1, 2))
    return out_full[:G]
